# baseline (speedup 1.0000x reference)
import functools
import math

import jax
import jax.numpy as jnp
import numpy as np
from jax import lax
from jax.experimental import pallas as pl
from jax.experimental.pallas import tpu as pltpu
from jax.experimental.pallas import tpu_sc as plsc

_NB = (24, 12, 16)
_PI = math.pi

_B, _T, _C = 16384, 512, 3
_PLANE = _B * _T
_F = _PLANE * _C
_MW = (_PLANE // 32) * 2

_NCORES, _NSUB = 2, 16
_NWORK = _NCORES * _NSUB
_SLAB = _PLANE // _NWORK

_CHUNK = 8192
_NCH = _SLAB // _CHUNK
_UNROLL = 32
_NVEC = _CHUNK // (16 * _UNROLL)
_NBUF = 2
_CHUNKW = _CHUNK // 32


def _sc_body(ang_hbm, msk_hbm, q_hbm, i_hbm,
             ang_v, msk_v, q_v, i_v,
             ain0, ain1, min0, min1, qout0, qout1, iout0, iout1):
    ain, min_ = (ain0, ain1), (min0, min1)
    qout, iout = (qout0, qout1), (iout0, iout1)
    wid = lax.axis_index("s") * _NCORES + lax.axis_index("c")
    sbase = wid * _SLAB

    for ci in range(3):
        n = _NB[ci]
        inv = float(np.float32(n / (2 * _PI)))
        halfn = float(np.float32(n * 0.5))
        width = float(np.float32(2 * _PI / n))
        pbase = ci * _PLANE + sbase
        mpbase = ci * (_PLANE // 32) + wid * (_SLAB // 32)
        has_mask = ci < 2

        def start_in(k, b, pbase=pbase, mpbase=mpbase, has_mask=has_mask):
            off = pbase + k * _CHUNK
            pltpu.async_copy(ang_hbm.at[pl.ds(off, _CHUNK)], ang_v.at[b], ain[b])
            if has_mask:
                offw = mpbase + k * _CHUNKW
                pltpu.async_copy(msk_hbm.at[pl.ds(offw, _CHUNKW)],
                                 msk_v.at[b, pl.ds(0, _CHUNKW)], min_[b])

        def wait_in(k, b, pbase=pbase, mpbase=mpbase, has_mask=has_mask):
            off = pbase + k * _CHUNK
            pltpu.make_async_copy(ang_hbm.at[pl.ds(off, _CHUNK)], ang_v.at[b],
                                  ain[b]).wait()
            if has_mask:
                offw = mpbase + k * _CHUNKW
                pltpu.make_async_copy(msk_hbm.at[pl.ds(offw, _CHUNKW)],
                                      msk_v.at[b, pl.ds(0, _CHUNKW)],
                                      min_[b]).wait()

        def start_out(k, b, pbase=pbase):
            off = pbase + k * _CHUNK
            pltpu.async_copy(q_v.at[b], q_hbm.at[pl.ds(off, _CHUNK)], qout[b])
            pltpu.async_copy(i_v.at[b], i_hbm.at[pl.ds(off, _CHUNK)], iout[b])

        def wait_out(k, b, pbase=pbase):
            off = pbase + k * _CHUNK
            pltpu.make_async_copy(q_v.at[b], q_hbm.at[pl.ds(off, _CHUNK)],
                                  qout[b]).wait()
            pltpu.make_async_copy(i_v.at[b], i_hbm.at[pl.ds(off, _CHUNK)],
                                  iout[b]).wait()

        def compute(b, ci=ci, n=n, inv=inv, halfn=halfn, width=width):
            def vec(v, c2):
                if ci < 2:
                    mwords = msk_v[b, pl.ds(v * 16, 16)]
                for u in range(_UNROLL):
                    o = v * (16 * _UNROLL) + u * 16
                    a = ang_v[b, pl.ds(o, 16)]
                    t = a * inv + halfn
                    i = jnp.minimum(t.astype(jnp.int32), n - 1)
                    q = (i.astype(jnp.float32) + 0.5) * width - _PI
                    if ci < 2:
                        m = (mwords & int(np.int32(np.uint32(1 << u)))) != 0
                        q = jnp.where(m, 0.0, q)
                        i = jnp.where(m, n, i)
                    q_v[b, pl.ds(o, 16)] = q
                    i_v[b, pl.ds(o, 16)] = i
                return c2

            lax.fori_loop(0, _NVEC, vec, 0)

        start_in(0, 0)

        def pair(g, carry):
            for b in range(_NBUF):
                k = g + b
                nxt = k + 1

                @pl.when(nxt < _NCH)
                def _():
                    start_in(nxt, 1 - b)

                wait_in(k, b)

                @pl.when(k >= _NBUF)
                def _():
                    wait_out(k - _NBUF, b)

                compute(b)
                start_out(k, b)
            return carry

        lax.fori_loop(0, _NCH // _NBUF, lambda g, c: pair(g * _NBUF, c), 0)
        wait_out(_NCH - 2, 0)
        wait_out(_NCH - 1, 1)


_mesh = plsc.VectorSubcoreMesh(core_axis_name="c", subcore_axis_name="s",
                               num_cores=_NCORES, num_subcores=_NSUB)

_sc_call = functools.partial(
    pl.kernel,
    compiler_params=pltpu.CompilerParams(needs_layout_passes=False),
    out_type=(jax.ShapeDtypeStruct((_F,), jnp.float32),
              jax.ShapeDtypeStruct((_F,), jnp.int32)),
    mesh=_mesh,
    scratch_types=[
        pltpu.VMEM((_NBUF, _CHUNK), jnp.float32),
        pltpu.VMEM((_NBUF, _CHUNKW), jnp.int32),
        pltpu.VMEM((_NBUF, _CHUNK), jnp.float32),
        pltpu.VMEM((_NBUF, _CHUNK), jnp.int32),
    ] + [pltpu.SemaphoreType.DMA] * 8,
)(_sc_body)


def kernel(angles, null_mask):
    a = jnp.transpose(angles, (2, 0, 1))
    a = a.reshape(3, 2048, 8, 4, 128).transpose(0, 1, 3, 2, 4).reshape(_F)
    mp = jnp.transpose(null_mask, (2, 0, 1))
    mp = mp.reshape(2, 2048, 8, 4, 128).transpose(0, 1, 3, 2, 4)
    bits = mp.reshape(2, _PLANE // 512, 32, 16).astype(jnp.uint32)
    m = jnp.sum(bits << jnp.arange(32, dtype=jnp.uint32)[:, None], axis=2)
    m = m.reshape(_MW).view(jnp.int32)

    q_flat, i_flat = _sc_call(a, m)

    q = q_flat.reshape(3, 2048, 4, 8, 128).transpose(0, 1, 3, 2, 4)
    q = q.reshape(3, _B, _T).transpose(1, 2, 0)
    ii = i_flat.reshape(3, 2048, 4, 8, 128).transpose(0, 1, 3, 2, 4)
    ii = ii.reshape(3, _B, _T).transpose(1, 2, 0)
    return (q, ii)


# speedup vs baseline: 4.5060x; 4.5060x over previous
import functools
import math

import jax
import jax.numpy as jnp
import numpy as np
from jax import lax
from jax.experimental import pallas as pl
from jax.experimental.pallas import tpu as pltpu
from jax.experimental.pallas import tpu_sc as plsc

_NB = (24, 12, 16)
_PI = math.pi

_B, _T, _C = 16384, 512, 3
_PLANE = _B * _T
_F = _PLANE * _C
_M = _PLANE * 2

_NCORES, _NSUB = 2, 16
_NWORK = _NCORES * _NSUB
_SLAB = _PLANE // _NWORK

_CHUNK = 8192
_NCH = _SLAB // _CHUNK
_UNROLL = 8
_NVEC = _CHUNK // (16 * _UNROLL)
_NBUF = 2


def _sc_body(ang_hbm, msk_hbm, q_hbm, i_hbm,
             ang_v, msk_v, q_v, i_v,
             ain0, ain1, min0, min1, qout0, qout1, iout0, iout1):
    ain, min_ = (ain0, ain1), (min0, min1)
    qout, iout = (qout0, qout1), (iout0, iout1)
    wid = lax.axis_index("s") * _NCORES + lax.axis_index("c")
    sbase = wid * _SLAB

    for ci in range(3):
        n = _NB[ci]
        inv = float(np.float32(n / (2 * _PI)))
        halfn = float(np.float32(n * 0.5))
        width = float(np.float32(2 * _PI / n))
        pbase = ci * _PLANE + sbase
        has_mask = ci < 2

        def start_in(k, b, pbase=pbase, has_mask=has_mask):
            off = pbase + k * _CHUNK
            pltpu.async_copy(ang_hbm.at[pl.ds(off, _CHUNK)], ang_v.at[b], ain[b])
            if has_mask:
                pltpu.async_copy(msk_hbm.at[pl.ds(off, _CHUNK)], msk_v.at[b],
                                 min_[b])

        def wait_in(k, b, pbase=pbase, has_mask=has_mask):
            off = pbase + k * _CHUNK
            pltpu.make_async_copy(ang_hbm.at[pl.ds(off, _CHUNK)], ang_v.at[b],
                                  ain[b]).wait()
            if has_mask:
                pltpu.make_async_copy(msk_hbm.at[pl.ds(off, _CHUNK)],
                                      msk_v.at[b], min_[b]).wait()

        def start_out(k, b, pbase=pbase):
            off = pbase + k * _CHUNK
            pltpu.async_copy(q_v.at[b], q_hbm.at[pl.ds(off, _CHUNK)], qout[b])
            pltpu.async_copy(i_v.at[b], i_hbm.at[pl.ds(off, _CHUNK)], iout[b])

        def wait_out(k, b, pbase=pbase):
            off = pbase + k * _CHUNK
            pltpu.make_async_copy(q_v.at[b], q_hbm.at[pl.ds(off, _CHUNK)],
                                  qout[b]).wait()
            pltpu.make_async_copy(i_v.at[b], i_hbm.at[pl.ds(off, _CHUNK)],
                                  iout[b]).wait()

        def compute(b, ci=ci, n=n, inv=inv, halfn=halfn, width=width):
            def vec(v, c2):
                for u in range(_UNROLL):
                    o = v * (16 * _UNROLL) + u * 16
                    a = ang_v[b, pl.ds(o, 16)]
                    t = a * inv + halfn
                    i = jnp.minimum(t.astype(jnp.int32), n - 1)
                    q = (i.astype(jnp.float32) + 0.5) * width - _PI
                    if ci < 2:
                        m = msk_v[b, pl.ds(o, 16)] != 0
                        q = jnp.where(m, 0.0, q)
                        i = jnp.where(m, n, i)
                    q_v[b, pl.ds(o, 16)] = q
                    i_v[b, pl.ds(o, 16)] = i
                return c2

            lax.fori_loop(0, _NVEC, vec, 0)

        start_in(0, 0)

        def pair(g, carry):
            for b in range(_NBUF):
                k = g + b
                nxt = k + 1

                @pl.when(nxt < _NCH)
                def _():
                    start_in(nxt, 1 - b)

                wait_in(k, b)

                @pl.when(k >= _NBUF)
                def _():
                    wait_out(k - _NBUF, b)

                compute(b)
                start_out(k, b)
            return carry

        lax.fori_loop(0, _NCH // _NBUF, lambda g, c: pair(g * _NBUF, c), 0)
        wait_out(_NCH - 2, 0)
        wait_out(_NCH - 1, 1)


_mesh = plsc.VectorSubcoreMesh(core_axis_name="c", subcore_axis_name="s",
                               num_cores=_NCORES, num_subcores=_NSUB)

_sc_call = functools.partial(
    pl.kernel,
    compiler_params=pltpu.CompilerParams(needs_layout_passes=False),
    out_type=(jax.ShapeDtypeStruct((_F,), jnp.float32),
              jax.ShapeDtypeStruct((_F,), jnp.int32)),
    mesh=_mesh,
    scratch_types=[
        pltpu.VMEM((_NBUF, _CHUNK), jnp.float32),
        pltpu.VMEM((_NBUF, _CHUNK), jnp.int32),
        pltpu.VMEM((_NBUF, _CHUNK), jnp.float32),
        pltpu.VMEM((_NBUF, _CHUNK), jnp.int32),
    ] + [pltpu.SemaphoreType.DMA] * 8,
)(_sc_body)


def kernel(angles, null_mask):
    a = jnp.transpose(angles, (2, 0, 1))
    a = a.reshape(3, 2048, 8, 4, 128).transpose(0, 1, 3, 2, 4).reshape(_F)
    mi = null_mask.astype(jnp.int32)
    m = jnp.transpose(mi, (2, 0, 1))
    m = m.reshape(2, 2048, 8, 4, 128).transpose(0, 1, 3, 2, 4).reshape(_M)

    q_flat, i_flat = _sc_call(a, m)

    q = q_flat.reshape(3, 2048, 4, 8, 128).transpose(0, 1, 3, 2, 4)
    q = q.reshape(3, _B, _T).transpose(1, 2, 0)
    ii = i_flat.reshape(3, 2048, 4, 8, 128).transpose(0, 1, 3, 2, 4)
    ii = ii.reshape(3, _B, _T).transpose(1, 2, 0)
    return (q, ii)
